# baseline (speedup 1.0000x reference)
import functools

import jax
import jax.numpy as jnp
from jax import lax
from jax.experimental import pallas as pl
from jax.experimental.pallas import tpu as pltpu
from jax.experimental.pallas import tpu_sc as plsc

D_MODEL = 64
SCALE = 0.125

NC = 2
NS = 16
NW = NC * NS
LANES = 16
UNIT = 128
PITCH = 129
SPLIT = 524288


def _make_kernel(S, NB):
    B = S * NB
    b_per_w = B // NW
    n_units = b_per_w // UNIT
    units_per_s = NB // UNIT
    assert B % NW == 0 and b_per_w % UNIT == 0 and n_units % 2 == 0

    mesh = plsc.VectorSubcoreMesh(core_axis_name="c", subcore_axis_name="s")

    @functools.partial(
        pl.kernel,
        out_type=jax.ShapeDtypeStruct((S, D_MODEL, NB), jnp.float32),
        mesh=mesh,
        compiler_params=pltpu.CompilerParams(needs_layout_passes=False),
        scratch_types=[
            pltpu.VMEM((b_per_w,), jnp.int32),
            pltpu.VMEM((UNIT,), jnp.int32),
            pltpu.VMEM((UNIT,), jnp.int32),
            pltpu.VMEM((2, UNIT, 128), jnp.float32),
            pltpu.VMEM((UNIT * PITCH,), jnp.float32),
            pltpu.VMEM((2, D_MODEL, UNIT), jnp.float32),
            pltpu.SemaphoreType.DMA,
            pltpu.SemaphoreType.DMA,
            pltpu.SemaphoreType.DMA,
            pltpu.SemaphoreType.DMA,
        ],
    )
    def emb_kernel(xt_hbm, t2_hbm, out_hbm, idx_v, i2a, i2b, rows_v,
                   rows_p, st_v, g0, g1, o0, o1):
        gsems = (g0, g1)
        osems = (o0, o1)
        idx2s = (i2a, i2b)
        wid = lax.axis_index("s") * NC + lax.axis_index("c")
        base = wid * b_per_w
        u0 = wid * n_units

        pltpu.sync_copy(xt_hbm.at[pl.ds(base, b_per_w)], idx_v)

        iota16 = lax.iota(jnp.int32, LANES)

        def prep_gather(u, slot):
            @pl.loop(0, UNIT // LANES)
            def _(t):
                v = idx_v[pl.ds(u * UNIT + t * LANES, LANES)]
                idx2s[slot][pl.ds(t * LANES, LANES)] = jnp.where(
                    v >= SPLIT, v - SPLIT, v)
            pltpu.async_copy(t2_hbm.at[idx2s[slot]], rows_v.at[slot],
                             gsems[slot])

        def wait_gather(slot):
            pltpu.make_async_copy(t2_hbm.at[idx2s[slot]],
                                  rows_v.at[slot], gsems[slot]).wait()

        def start_store(u, slot):
            ug = u0 + u
            s = ug // units_per_s
            b0 = (ug % units_per_s) * UNIT
            pltpu.async_copy(st_v.at[slot],
                             out_hbm.at[s, :, pl.ds(b0, UNIT)], osems[slot])

        def wait_store(slot):
            pltpu.make_async_copy(st_v.at[slot],
                                  out_hbm.at[0, :, pl.ds(0, UNIT)],
                                  osems[slot]).wait()

        def select_transpose_scale(u, slot):
            @pl.loop(0, UNIT // 2)
            def _(r2):
                for rr in range(2):
                    for c in range(8):
                        v = rows_v[slot, r2 * 2 + rr, pl.ds(c * LANES, LANES)]
                        plsc.store_scatter(
                            rows_p,
                            [iota16 + ((r2 * 2 + rr) * PITCH + c * LANES)], v)

            @pl.loop(0, UNIT // LANES)
            def _(bg):
                half = (idx_v[pl.ds(u * UNIT + bg * LANES, LANES)]
                        >= SPLIT).astype(jnp.int32) << 6
                rv = (iota16 + bg * LANES) * PITCH + half
                for d0 in range(0, D_MODEL, 8):
                    vs = [plsc.load_gather(rows_p, [rv + (d0 + k)])
                          for k in range(8)]
                    for k in range(8):
                        st_v[slot, d0 + k, pl.ds(bg * LANES, LANES)] = (
                            vs[k] * SCALE)

        prep_gather(0, 0)

        @pl.loop(0, n_units, step=2)
        def _(i):
            @pl.when(i > 0)
            def _():
                wait_store(1)
            prep_gather(i + 1, 1)
            wait_gather(0)
            select_transpose_scale(i, 0)
            start_store(i, 0)

            @pl.when(i + 2 < n_units)
            def _():
                wait_store(0)
                prep_gather(i + 2, 0)
            wait_gather(1)
            select_transpose_scale(i + 1, 1)
            start_store(i + 1, 1)

        wait_store(0)
        wait_store(1)

    return emb_kernel


def _make_table_builder(V):
    CB = 2048
    n_blocks = SPLIT // CB
    last_in = (V + CB - 1) // CB - 1

    def body(a_ref, b_ref, o_ref):
        o_ref[:, 0:D_MODEL] = jnp.transpose(a_ref[...], (1, 0))
        o_ref[:, D_MODEL:2 * D_MODEL] = jnp.transpose(b_ref[...], (1, 0))

    return pl.pallas_call(
        body,
        grid=(n_blocks,),
        in_specs=[
            pl.BlockSpec((D_MODEL, CB), lambda i: (0, i)),
            pl.BlockSpec((D_MODEL, CB),
                         lambda i: (0, jnp.minimum(i + n_blocks, last_in))),
        ],
        out_specs=pl.BlockSpec((CB, 2 * D_MODEL), lambda i: (i, 0)),
        out_shape=jax.ShapeDtypeStruct((SPLIT, 2 * D_MODEL), jnp.float32),
    )


def kernel(x, table):
    nb, ns = x.shape
    nv, d = table.shape
    xt = jnp.transpose(x).reshape(-1).astype(jnp.int32)
    tt = jnp.transpose(table)
    t2 = _make_table_builder(nv)(tt, tt)
    o = _make_kernel(ns, nb)(xt, t2)
    return jnp.transpose(o, (2, 0, 1))


# speedup vs baseline: 1.1053x; 1.1053x over previous
import functools

import jax
import jax.numpy as jnp
from jax import lax
from jax.experimental import pallas as pl
from jax.experimental.pallas import tpu as pltpu
from jax.experimental.pallas import tpu_sc as plsc

D_MODEL = 64
SCALE = 0.125

NC = 2
NS = 16
NW = NC * NS
LANES = 16
UNIT = 256
SPLIT = 524288


def _make_kernel(S, NB):
    B = S * NB
    b_per_w = B // NW
    n_units = b_per_w // UNIT
    units_per_s = NB // UNIT
    assert B % NW == 0 and b_per_w % UNIT == 0 and n_units % 2 == 0

    mesh = plsc.VectorSubcoreMesh(core_axis_name="c", subcore_axis_name="s")

    @functools.partial(
        pl.kernel,
        out_type=jax.ShapeDtypeStruct((S, D_MODEL, NB), jnp.float32),
        mesh=mesh,
        compiler_params=pltpu.CompilerParams(needs_layout_passes=False),
        scratch_types=[
            pltpu.VMEM((b_per_w,), jnp.int32),
            pltpu.VMEM((UNIT,), jnp.int32),
            pltpu.VMEM((UNIT,), jnp.int32),
            pltpu.VMEM((2, UNIT, 128), jnp.float32),
            pltpu.VMEM((2, D_MODEL, UNIT), jnp.float32),
            pltpu.SemaphoreType.DMA,
            pltpu.SemaphoreType.DMA,
            pltpu.SemaphoreType.DMA,
            pltpu.SemaphoreType.DMA,
        ],
    )
    def emb_kernel(xt_hbm, t2_hbm, out_hbm, idx_v, i2a, i2b, rows_v,
                   st_v, g0, g1, o0, o1):
        gsems = (g0, g1)
        osems = (o0, o1)
        idx2s = (i2a, i2b)
        wid = lax.axis_index("s") * NC + lax.axis_index("c")
        base = wid * b_per_w
        u0 = wid * n_units

        pltpu.sync_copy(xt_hbm.at[pl.ds(base, b_per_w)], idx_v)

        iota16 = lax.iota(jnp.int32, LANES)

        def prep_gather(u, slot):
            @pl.loop(0, UNIT // LANES)
            def _(t):
                v = idx_v[pl.ds(u * UNIT + t * LANES, LANES)]
                idx2s[slot][pl.ds(t * LANES, LANES)] = jnp.where(
                    v >= SPLIT, v - SPLIT, v)
            pltpu.async_copy(t2_hbm.at[idx2s[slot]], rows_v.at[slot],
                             gsems[slot])

        def wait_gather(slot):
            pltpu.make_async_copy(t2_hbm.at[idx2s[slot]],
                                  rows_v.at[slot], gsems[slot]).wait()

        def start_store(u, slot):
            ug = u0 + u
            s = ug // units_per_s
            b0 = (ug % units_per_s) * UNIT
            pltpu.async_copy(st_v.at[slot],
                             out_hbm.at[s, :, pl.ds(b0, UNIT)], osems[slot])

        def wait_store(slot):
            pltpu.make_async_copy(st_v.at[slot],
                                  out_hbm.at[0, :, pl.ds(0, UNIT)],
                                  osems[slot]).wait()

        def select_transpose_scale(u, slot):
            @pl.loop(0, UNIT // LANES)
            def _(bg):
                half = (idx_v[pl.ds(u * UNIT + bg * LANES, LANES)]
                        >= SPLIT).astype(jnp.int32) << 6
                rv = iota16 + bg * LANES
                for d0 in range(0, D_MODEL, 16):
                    vs = [plsc.load_gather(rows_v.at[slot],
                                           [rv, half + (d0 + k)])
                          for k in range(16)]
                    for k in range(16):
                        st_v[slot, d0 + k, pl.ds(bg * LANES, LANES)] = (
                            vs[k] * SCALE)

        prep_gather(0, 0)

        @pl.loop(0, n_units, step=2)
        def _(i):
            @pl.when(i > 0)
            def _():
                wait_store(1)
            prep_gather(i + 1, 1)
            wait_gather(0)
            select_transpose_scale(i, 0)
            start_store(i, 0)

            @pl.when(i + 2 < n_units)
            def _():
                wait_store(0)
                prep_gather(i + 2, 0)
            wait_gather(1)
            select_transpose_scale(i + 1, 1)
            start_store(i + 1, 1)

        wait_store(0)
        wait_store(1)

    return emb_kernel


def _make_table_builder(V):
    CB = 2048
    n_blocks = SPLIT // CB
    last_in = (V + CB - 1) // CB - 1

    def body(a_ref, b_ref, o_ref):
        o_ref[:, 0:D_MODEL] = jnp.transpose(a_ref[...], (1, 0))
        o_ref[:, D_MODEL:2 * D_MODEL] = jnp.transpose(b_ref[...], (1, 0))

    return pl.pallas_call(
        body,
        grid=(n_blocks,),
        in_specs=[
            pl.BlockSpec((D_MODEL, CB), lambda i: (0, i)),
            pl.BlockSpec((D_MODEL, CB),
                         lambda i: (0, jnp.minimum(i + n_blocks, last_in))),
        ],
        out_specs=pl.BlockSpec((CB, 2 * D_MODEL), lambda i: (i, 0)),
        out_shape=jax.ShapeDtypeStruct((SPLIT, 2 * D_MODEL), jnp.float32),
    )


def kernel(x, table):
    nb, ns = x.shape
    nv, d = table.shape
    xt = jnp.transpose(x).reshape(-1).astype(jnp.int32)
    tt = jnp.transpose(table)
    t2 = _make_table_builder(nv)(tt, tt)
    o = _make_kernel(ns, nb)(xt, t2)
    return jnp.transpose(o, (2, 0, 1))
